# Initial kernel scaffold; baseline (speedup 1.0000x reference)
import functools

import jax
import jax.numpy as jnp
from jax import lax
from jax.experimental import pallas as pl
from jax.experimental.pallas import tpu as pltpu
from jax.experimental.pallas import tpu_sc as plsc

N_NODES = 10000
N_EDGES = 320000
D = 128
N_GRAPHS = 64
D_OUT = 100

NC = 2
NS = 16
NW = NC * NS
EW = N_EDGES // NW
K = 80
N_PER_SUB = N_NODES // NS
DEG_W = 16

R = 2000
GRID = N_NODES // R

_sc_mesh = plsc.VectorSubcoreMesh(core_axis_name="c", subcore_axis_name="s")


@functools.partial(
    pl.kernel,
    out_type=jax.ShapeDtypeStruct((NC, N_NODES, DEG_W), jnp.float32),
    mesh=_sc_mesh,
    scratch_types=[
        pltpu.VMEM((K,), jnp.int32),
        pltpu.VMEM((K, DEG_W), jnp.float32),
        pltpu.VMEM_SHARED((N_NODES, DEG_W), jnp.float32),
    ],
)
def _deg_kernel(dst_hbm, ones_hbm, zeros_hbm, out_hbm, didx, ones_v, acc):
    c = lax.axis_index("c")
    s = lax.axis_index("s")
    wid = s * NC + c
    pltpu.sync_copy(zeros_hbm, acc.at[pl.ds(s * N_PER_SUB, N_PER_SUB)])
    pltpu.sync_copy(ones_hbm, ones_v)
    plsc.subcore_barrier()
    base = wid * EW

    def body(i, carry):
        pltpu.sync_copy(dst_hbm.at[pl.ds(base + i * K, K)], didx)
        pltpu.sync_copy(ones_v, acc.at[didx], add=True)
        return carry

    lax.fori_loop(0, EW // K, body, 0)
    plsc.subcore_barrier()
    pltpu.sync_copy(acc.at[pl.ds(s * N_PER_SUB, N_PER_SUB)],
                    out_hbm.at[c, pl.ds(s * N_PER_SUB, N_PER_SUB)])


@functools.partial(
    pl.kernel,
    out_type=jax.ShapeDtypeStruct((NC, N_NODES, D), jnp.float32),
    mesh=_sc_mesh,
    scratch_types=[
        pltpu.VMEM((K,), jnp.int32),
        pltpu.VMEM((K,), jnp.int32),
        pltpu.VMEM((K, D), jnp.float32),
        pltpu.VMEM_SHARED((N_NODES, D), jnp.float32),
        pltpu.SemaphoreType.DMA,
    ],
)
def _agg_kernel(hw_hbm, src_hbm, dst_hbm, zeros_hbm, out_hbm,
                sidx, didx, rows, acc, sem):
    c = lax.axis_index("c")
    s = lax.axis_index("s")
    wid = s * NC + c
    pltpu.sync_copy(zeros_hbm, acc.at[pl.ds(s * N_PER_SUB, N_PER_SUB)])
    plsc.subcore_barrier()
    base = wid * EW

    def body(i, carry):
        off = base + i * K
        pltpu.sync_copy(src_hbm.at[pl.ds(off, K)], sidx)
        pltpu.sync_copy(dst_hbm.at[pl.ds(off, K)], didx)
        pltpu.async_copy(hw_hbm.at[sidx], rows, sem).wait()
        pltpu.sync_copy(rows, acc.at[didx], add=True)
        return carry

    lax.fori_loop(0, EW // K, body, 0)
    plsc.subcore_barrier()
    pltpu.sync_copy(acc.at[pl.ds(s * N_PER_SUB, N_PER_SUB)],
                    out_hbm.at[c, pl.ds(s * N_PER_SUB, N_PER_SUB)])


def _dinv_block(degp_ref):
    deg = degp_ref[0, :, 0:1] + degp_ref[1, :, 0:1] + 1.0
    return lax.rsqrt(deg)


def _enc_body(x_ref, degp_ref, W1_ref, b1_ref, W2_ref, b2_ref, Wc1_ref, out_ref):
    h = jnp.maximum(jnp.dot(x_ref[...], W1_ref[...],
                            preferred_element_type=jnp.float32) + b1_ref[...], 0.0)
    h = jnp.dot(h, W2_ref[...], preferred_element_type=jnp.float32) + b2_ref[...]
    hw = jnp.dot(h, Wc1_ref[...], preferred_element_type=jnp.float32)
    out_ref[...] = hw * _dinv_block(degp_ref)


def _mid_body(accp_ref, hws_ref, degp_ref, bc1_ref, Wc2_ref, out_ref):
    dinv = _dinv_block(degp_ref)
    out1 = (accp_ref[0] + accp_ref[1] + hws_ref[...]) * dinv + bc1_ref[...]
    out_ref[...] = jnp.dot(out1, Wc2_ref[...],
                           preferred_element_type=jnp.float32) * dinv


def _fin_body(accp_ref, hws_ref, degp_ref, bc2_ref, batch_ref, Wcls_ref, bcls_ref,
              h_ref, logits_ref, seg_acc, cnt_acc):
    i = pl.program_id(0)
    dinv = _dinv_block(degp_ref)
    h = (accp_ref[0] + accp_ref[1] + hws_ref[...]) * dinv + bc2_ref[...]
    h_ref[...] = h
    gids = lax.broadcasted_iota(jnp.int32, (N_GRAPHS, R), 0)
    onehot = (batch_ref[0] == gids).astype(jnp.float32)
    part = jnp.dot(onehot, h, preferred_element_type=jnp.float32)
    cnt = jnp.broadcast_to(jnp.sum(onehot, axis=1, keepdims=True), (N_GRAPHS, D))

    @pl.when(i == 0)
    def _():
        seg_acc[...] = part
        cnt_acc[...] = cnt

    @pl.when(i > 0)
    def _():
        seg_acc[...] += part
        cnt_acc[...] += cnt

    @pl.when(i == pl.num_programs(0) - 1)
    def _():
        hg = seg_acc[...] / jnp.maximum(cnt_acc[...], 1.0)
        logits_ref[...] = jnp.dot(hg, Wcls_ref[...],
                                  preferred_element_type=jnp.float32) + bcls_ref[...]


def _row_spec(width):
    return pl.BlockSpec((R, width), lambda i: (i, 0))


def _part_spec(width):
    return pl.BlockSpec((NC, R, width), lambda i: (0, i, 0))


def _full_spec(shape):
    nd = len(shape)
    return pl.BlockSpec(shape, lambda i, _nd=nd: (0,) * _nd)


_enc_call = pl.pallas_call(
    _enc_body,
    grid=(GRID,),
    in_specs=[
        _row_spec(D), _part_spec(DEG_W),
        _full_spec((D, D)), _full_spec((1, D)),
        _full_spec((D, D)), _full_spec((1, D)),
        _full_spec((D, D)),
    ],
    out_specs=_row_spec(D),
    out_shape=jax.ShapeDtypeStruct((N_NODES, D), jnp.float32),
)

_mid_call = pl.pallas_call(
    _mid_body,
    grid=(GRID,),
    in_specs=[
        _part_spec(D), _row_spec(D), _part_spec(DEG_W),
        _full_spec((1, D)), _full_spec((D, D)),
    ],
    out_specs=_row_spec(D),
    out_shape=jax.ShapeDtypeStruct((N_NODES, D), jnp.float32),
)

_fin_call = pl.pallas_call(
    _fin_body,
    grid=(GRID,),
    in_specs=[
        _part_spec(D), _row_spec(D), _part_spec(DEG_W),
        _full_spec((1, D)),
        pl.BlockSpec((1, 1, R), lambda i: (i, 0, 0)),
        _full_spec((D, D)), _full_spec((1, D)),
    ],
    out_specs=[_row_spec(D), _full_spec((N_GRAPHS, D))],
    out_shape=[
        jax.ShapeDtypeStruct((N_NODES, D), jnp.float32),
        jax.ShapeDtypeStruct((N_GRAPHS, D), jnp.float32),
    ],
    scratch_shapes=[
        pltpu.VMEM((N_GRAPHS, D), jnp.float32),
        pltpu.VMEM((N_GRAPHS, D), jnp.float32),
    ],
)


def kernel(x, edge_index, batch, W1, b1, W2, b2, Wc1, bc1, Wc2, bc2, Wcls, bcls):
    src = edge_index[0]
    dst = edge_index[1]
    zeros16 = jnp.zeros((N_PER_SUB, DEG_W), jnp.float32)
    ones16 = jnp.ones((K, DEG_W), jnp.float32)
    zeros128 = jnp.zeros((N_PER_SUB, D), jnp.float32)

    degp = _deg_kernel(dst, ones16, zeros16)

    hw1s = _enc_call(x, degp, W1, b1.reshape(1, D), W2, b2.reshape(1, D), Wc1)
    acc1 = _agg_kernel(hw1s, src, dst, zeros128)
    hw2s = _mid_call(acc1, hw1s, degp, bc1.reshape(1, D), Wc2)
    acc2 = _agg_kernel(hw2s, src, dst, zeros128)

    Wcls_p = jnp.pad(Wcls, ((0, 0), (0, D - D_OUT)))
    bcls_p = jnp.pad(bcls, (0, D - D_OUT)).reshape(1, D)
    batch_r = batch.reshape(GRID, 1, R)
    h_out, logits_p = _fin_call(acc2, hw2s, degp, bc2.reshape(1, D),
                                batch_r, Wcls_p, bcls_p)
    return (logits_p[:, :D_OUT], h_out)


# speedup vs baseline: 13.4094x; 13.4094x over previous
import functools

import jax
import jax.numpy as jnp
from jax import lax
from jax.experimental import pallas as pl
from jax.experimental.pallas import tpu as pltpu
from jax.experimental.pallas import tpu_sc as plsc

N_NODES = 10000
N_EDGES = 320000
D = 128
N_GRAPHS = 64
D_OUT = 100

NC = 2
NS = 16
NW = NC * NS
EW = N_EDGES // NW
K = 80
CH = 624
TAIL = N_NODES - NS * CH

R = 2000
GRID = N_NODES // R


@functools.cache
def _sc_kernels():
    mesh = plsc.VectorSubcoreMesh(core_axis_name="c", subcore_axis_name="s",
                                  num_cores=NC, num_subcores=NS)

    @functools.partial(
        pl.kernel,
        out_type=jax.ShapeDtypeStruct((NW, N_NODES), jnp.float32),
        mesh=mesh,
        compiler_params=pltpu.CompilerParams(needs_layout_passes=False),
        scratch_types=[
            pltpu.VMEM((K,), jnp.int32),
            pltpu.VMEM((16,), jnp.float32),
            pltpu.VMEM((N_NODES,), jnp.float32),
        ],
    )
    def deg_kernel(dst_hbm, out_hbm, didx, ones_v, tab):
        c = lax.axis_index("c")
        s = lax.axis_index("s")
        wid = s * NC + c
        ones_v[...] = jnp.full((16,), 1.0, jnp.float32)
        zv = ones_v[...] * 0.0

        def zbody(i, carry):
            tab[pl.ds(i * 16, 16)] = zv
            return carry

        lax.fori_loop(0, N_NODES // 16, zbody, 0)
        base = wid * EW

        def body(i, carry):
            pltpu.sync_copy(dst_hbm.at[pl.ds(base + i * K, K)], didx)
            ov = ones_v[...]
            for j in range(K // 16):
                dvec = didx[pl.ds(j * 16, 16)]
                plsc.addupdate_scatter(tab, [dvec], ov)
            return carry

        lax.fori_loop(0, EW // K, body, 0)
        pltpu.sync_copy(tab, out_hbm.at[wid])

    @functools.partial(
        pl.kernel,
        out_type=jax.ShapeDtypeStruct((NC, N_NODES, D), jnp.float32),
        mesh=mesh,
        scratch_types=[
            pltpu.VMEM((K,), jnp.int32),
            pltpu.VMEM((K,), jnp.int32),
            pltpu.VMEM((K, D), jnp.float32),
            pltpu.VMEM_SHARED((N_NODES, D), jnp.float32),
            pltpu.SemaphoreType.DMA,
        ],
    )
    def agg_kernel(hw_hbm, src_hbm, dst_hbm, zeros_hbm, out_hbm,
                   sidx, didx, rows, acc, sem):
        c = lax.axis_index("c")
        s = lax.axis_index("s")
        wid = s * NC + c
        pltpu.sync_copy(zeros_hbm.at[pl.ds(0, CH)], acc.at[pl.ds(s * CH, CH)])

        @pl.when(s == NS - 1)
        def _():
            pltpu.sync_copy(zeros_hbm.at[pl.ds(0, TAIL)],
                            acc.at[pl.ds(NS * CH, TAIL)])

        plsc.subcore_barrier()
        base = wid * EW

        def body(i, carry):
            off = base + i * K
            pltpu.sync_copy(src_hbm.at[pl.ds(off, K)], sidx)
            pltpu.sync_copy(dst_hbm.at[pl.ds(off, K)], didx)
            pltpu.async_copy(hw_hbm.at[sidx], rows, sem).wait()
            pltpu.sync_copy(rows, acc.at[didx], add=True)
            return carry

        lax.fori_loop(0, EW // K, body, 0)
        plsc.subcore_barrier()
        pltpu.sync_copy(acc.at[pl.ds(s * CH, CH)],
                        out_hbm.at[c, pl.ds(s * CH, CH)])

        @pl.when(s == NS - 1)
        def _():
            pltpu.sync_copy(acc.at[pl.ds(NS * CH, TAIL)],
                            out_hbm.at[c, pl.ds(NS * CH, TAIL)])

    return deg_kernel, agg_kernel


def _dinv_block(degw_ref):
    ones_col = jnp.ones((NW, 1), jnp.float32)
    deg = jnp.dot(degw_ref[...], ones_col,
                  preferred_element_type=jnp.float32) + 1.0
    return lax.rsqrt(deg)


def _enc_body(x_ref, degw_ref, W1_ref, b1_ref, W2_ref, b2_ref, Wc1_ref, out_ref):
    h = jnp.maximum(jnp.dot(x_ref[...], W1_ref[...],
                            preferred_element_type=jnp.float32) + b1_ref[...], 0.0)
    h = jnp.dot(h, W2_ref[...], preferred_element_type=jnp.float32) + b2_ref[...]
    hw = jnp.dot(h, Wc1_ref[...], preferred_element_type=jnp.float32)
    out_ref[...] = hw * _dinv_block(degw_ref)


def _mid_body(accp_ref, hws_ref, degw_ref, bc1_ref, Wc2_ref, out_ref):
    dinv = _dinv_block(degw_ref)
    out1 = (accp_ref[0] + accp_ref[1] + hws_ref[...]) * dinv + bc1_ref[...]
    out_ref[...] = jnp.dot(out1, Wc2_ref[...],
                           preferred_element_type=jnp.float32) * dinv


def _fin_body(accp_ref, hws_ref, degw_ref, bc2_ref, batch_ref, Wcls_ref, bcls_ref,
              h_ref, logits_ref, seg_acc, cnt_acc):
    i = pl.program_id(0)
    dinv = _dinv_block(degw_ref)
    h = (accp_ref[0] + accp_ref[1] + hws_ref[...]) * dinv + bc2_ref[...]
    h_ref[...] = h
    gids = lax.broadcasted_iota(jnp.int32, (N_GRAPHS, R), 0)
    onehot = (batch_ref[0] == gids).astype(jnp.float32)
    part = jnp.dot(onehot, h, preferred_element_type=jnp.float32)
    cnt = jnp.broadcast_to(jnp.sum(onehot, axis=1, keepdims=True), (N_GRAPHS, D))

    @pl.when(i == 0)
    def _():
        seg_acc[...] = part
        cnt_acc[...] = cnt

    @pl.when(i > 0)
    def _():
        seg_acc[...] += part
        cnt_acc[...] += cnt

    @pl.when(i == pl.num_programs(0) - 1)
    def _():
        hg = seg_acc[...] / jnp.maximum(cnt_acc[...], 1.0)
        logits_ref[...] = jnp.dot(hg, Wcls_ref[...],
                                  preferred_element_type=jnp.float32) + bcls_ref[...]


def _row_spec(width):
    return pl.BlockSpec((R, width), lambda i: (i, 0))


def _degw_spec():
    return pl.BlockSpec((R, NW), lambda i: (i, 0))


def _part_spec(width):
    return pl.BlockSpec((NC, R, width), lambda i: (0, i, 0))


def _full_spec(shape):
    nd = len(shape)
    return pl.BlockSpec(shape, lambda i, _nd=nd: (0,) * _nd)


_enc_call = pl.pallas_call(
    _enc_body,
    grid=(GRID,),
    in_specs=[
        _row_spec(D), _degw_spec(),
        _full_spec((D, D)), _full_spec((1, D)),
        _full_spec((D, D)), _full_spec((1, D)),
        _full_spec((D, D)),
    ],
    out_specs=_row_spec(D),
    out_shape=jax.ShapeDtypeStruct((N_NODES, D), jnp.float32),
)

_mid_call = pl.pallas_call(
    _mid_body,
    grid=(GRID,),
    in_specs=[
        _part_spec(D), _row_spec(D), _degw_spec(),
        _full_spec((1, D)), _full_spec((D, D)),
    ],
    out_specs=_row_spec(D),
    out_shape=jax.ShapeDtypeStruct((N_NODES, D), jnp.float32),
)

_fin_call = pl.pallas_call(
    _fin_body,
    grid=(GRID,),
    in_specs=[
        _part_spec(D), _row_spec(D), _degw_spec(),
        _full_spec((1, D)),
        pl.BlockSpec((1, 1, R), lambda i: (i, 0, 0)),
        _full_spec((D, D)), _full_spec((1, D)),
    ],
    out_specs=[_row_spec(D), _full_spec((N_GRAPHS, D))],
    out_shape=[
        jax.ShapeDtypeStruct((N_NODES, D), jnp.float32),
        jax.ShapeDtypeStruct((N_GRAPHS, D), jnp.float32),
    ],
    scratch_shapes=[
        pltpu.VMEM((N_GRAPHS, D), jnp.float32),
        pltpu.VMEM((N_GRAPHS, D), jnp.float32),
    ],
)


def kernel(x, edge_index, batch, W1, b1, W2, b2, Wc1, bc1, Wc2, bc2, Wcls, bcls):
    src = edge_index[0]
    dst = edge_index[1]
    zeros128 = jnp.zeros((CH, D), jnp.float32)

    _deg_kernel, _agg_kernel = _sc_kernels()
    degw = _deg_kernel(dst).T

    hw1s = _enc_call(x, degw, W1, b1.reshape(1, D), W2, b2.reshape(1, D), Wc1)
    acc1 = _agg_kernel(hw1s, src, dst, zeros128)
    hw2s = _mid_call(acc1, hw1s, degw, bc1.reshape(1, D), Wc2)
    acc2 = _agg_kernel(hw2s, src, dst, zeros128)

    Wcls_p = jnp.pad(Wcls, ((0, 0), (0, D - D_OUT)))
    bcls_p = jnp.pad(bcls, (0, D - D_OUT)).reshape(1, D)
    batch_r = batch.reshape(GRID, 1, R)
    h_out, logits_p = _fin_call(acc2, hw2s, degw, bc2.reshape(1, D),
                                batch_r, Wcls_p, bcls_p)
    return (logits_p[:, :D_OUT], h_out)
